# Initial kernel scaffold; baseline (speedup 1.0000x reference)
import jax
import jax.numpy as jnp
from jax.experimental import pallas as pl


def kernel(X, edge_index, bn1_gamma, bn1_beta, bn2_gamma, bn2_beta, lstm1_W, lstm1_U, lstm1_b, lstm2_W, lstm2_U, lstm2_b, lin_W, lin_b):
    raise NotImplementedError("write your pallas kernel here")


# speedup vs baseline: 5.8898x; 5.8898x over previous
import functools

import jax
import jax.numpy as jnp
from jax import lax
from jax.experimental import pallas as pl
from jax.experimental.pallas import tpu as pltpu
from jax.experimental.pallas import tpu_sc as plsc

_BN_EPS = 1e-3

_N = 10000
_E = 160000
_F = 128
_W = 6
_H = 52

_NC = 2
_NS = 16
_TPC = _W // _NC
_EPT = _E // _NS
_K = 80
_NCH = _EPT // _K
_RC = 80
_NCHN = _N // _RC
_CPT = (_NCHN + _NS - 1) // _NS

_TB = 200
_G = 128


def _sc_mpnn_body(x_hbm, src_hbm, dst_hbm, g1_hbm, b1_hbm, g2_hbm, b2_hbm,
                  h1_hbm, h2_hbm,
                  acc, idx_raw, idx_g, rows, zbuf, cbuf, pg, pb, sem):
    c = lax.axis_index("c")
    s = lax.axis_index("s")
    t0 = c * _TPC
    ebase = s * _EPT

    zero16 = jnp.zeros((16,), jnp.float32)

    def _zrow(r, carry):
        for j in range(_F // 16):
            zbuf[r, pl.ds(j * 16, 16)] = zero16
        return carry

    lax.fori_loop(0, _RC, _zrow, 0)

    def _for_my_node_chunks(fn):
        def body(ci, carry):
            cid = s + ci * _NS

            @pl.when(cid < _NCHN)
            def _():
                fn(cid * _RC)

            return carry

        lax.fori_loop(0, _CPT, body, 0)

    def _zero_acc_slice():
        _for_my_node_chunks(
            lambda base: pltpu.sync_copy(zbuf, acc.at[pl.ds(base, _RC)]))

    def _edge_round(table_hbm, toffs, t):
        eoffs = t * _E + ebase

        def _chunk(k, carry):
            off = eoffs + k * _K
            pltpu.sync_copy(src_hbm.at[pl.ds(off, _K)], idx_raw)
            for j in range(_K // 16):
                idx_g[pl.ds(j * 16, 16)] = idx_raw[pl.ds(j * 16, 16)] + toffs
            pltpu.async_copy(table_hbm.at[idx_g], rows, sem).wait()
            pltpu.sync_copy(dst_hbm.at[pl.ds(off, _K)], idx_raw)
            pltpu.sync_copy(rows, acc.at[idx_raw], add=True)
            return carry

        lax.fori_loop(0, _NCH, _chunk, 0)

    def _ew_round(gx_hbm, bx_hbm, out_hbm, toffs, t):
        pltpu.sync_copy(gx_hbm.at[pl.ds(t * _F, _F)], pg)
        pltpu.sync_copy(bx_hbm.at[pl.ds(t * _F, _F)], pb)

        def _one(base):
            pltpu.sync_copy(acc.at[pl.ds(base, _RC)], cbuf)

            def _row(r, carry):
                for j in range(_F // 16):
                    v = cbuf[r, pl.ds(j * 16, 16)]
                    v = jnp.maximum(v, 0.0) * pg[pl.ds(j * 16, 16)] \
                        + pb[pl.ds(j * 16, 16)]
                    cbuf[r, pl.ds(j * 16, 16)] = v
                return carry

            lax.fori_loop(0, _RC, _row, 0)
            pltpu.sync_copy(cbuf, out_hbm.at[pl.ds(toffs + base, _RC)])
            pltpu.sync_copy(zbuf, acc.at[pl.ds(base, _RC)])

        _for_my_node_chunks(_one)

    _zero_acc_slice()
    for j in range(_TPC):
        t = t0 + j
        toffs = t * _N
        plsc.subcore_barrier()
        _edge_round(x_hbm, toffs, t)
        plsc.subcore_barrier()
        _ew_round(g1_hbm, b1_hbm, h1_hbm, toffs, t)
        plsc.subcore_barrier()
        _edge_round(h1_hbm, toffs, t)
        plsc.subcore_barrier()
        _ew_round(g2_hbm, b2_hbm, h2_hbm, toffs, t)


def _sc_mpnn(x_flat, src, dst, g1, b1, g2, b2):
    mesh = plsc.VectorSubcoreMesh(core_axis_name="c", subcore_axis_name="s",
                                  num_cores=_NC, num_subcores=_NS)
    f = pl.kernel(
        _sc_mpnn_body,
        out_type=(
            jax.ShapeDtypeStruct((_W * _N, _F), jnp.float32),
            jax.ShapeDtypeStruct((_W * _N, _F), jnp.float32),
        ),
        mesh=mesh,
        scratch_types=[
            pltpu.VMEM_SHARED((_N, _F), jnp.float32),
            pltpu.VMEM((_K,), jnp.int32),
            pltpu.VMEM((_K,), jnp.int32),
            pltpu.VMEM((_K, _F), jnp.float32),
            pltpu.VMEM((_RC, _F), jnp.float32),
            pltpu.VMEM((_RC, _F), jnp.float32),
            pltpu.VMEM((_F,), jnp.float32),
            pltpu.VMEM((_F,), jnp.float32),
            pltpu.SemaphoreType.DMA,
        ],
    )
    return f(x_flat, src, dst, g1, b1, g2, b2)


def _lstm_tc_body(seq_ref, w1_ref, u1_ref, b1_ref, w2_ref, b2_ref,
                  lw_ref, lb_ref, out_ref, xw_ref, carry_ref):
    i = pl.program_id(0)

    @pl.when(i == 0)
    def _():
        carry_ref[...] = jnp.zeros_like(carry_ref)

    x = seq_ref[...].reshape(_TB * 8, 2 * _F)
    xw_ref[...] = (
        jnp.dot(x, w1_ref[...], preferred_element_type=jnp.float32)
        + b1_ref[...]
    )

    def _step(t, carry):
        h1, c1, h2, c2 = carry
        xwt = xw_ref[pl.ds(pl.multiple_of(t * 8, 8), 8), :]
        z1 = xwt + jnp.dot(h1, u1_ref[...], preferred_element_type=jnp.float32)
        i1 = jax.nn.sigmoid(z1[:, 0 * _G:1 * _G])
        f1 = jax.nn.sigmoid(z1[:, 1 * _G:2 * _G])
        g1 = jnp.tanh(z1[:, 2 * _G:3 * _G])
        o1 = jax.nn.sigmoid(z1[:, 3 * _G:4 * _G])
        c1 = f1 * c1 + i1 * g1
        h1 = o1 * jnp.tanh(c1)
        hcat = jnp.concatenate([h1, h2], axis=1)
        z2 = (jnp.dot(hcat, w2_ref[...], preferred_element_type=jnp.float32)
              + b2_ref[...])
        i2 = jax.nn.sigmoid(z2[:, 0 * _G:1 * _G])
        f2 = jax.nn.sigmoid(z2[:, 1 * _G:2 * _G])
        g2 = jnp.tanh(z2[:, 2 * _G:3 * _G])
        o2 = jax.nn.sigmoid(z2[:, 3 * _G:4 * _G])
        c2 = f2 * c2 + i2 * g2
        h2 = o2 * jnp.tanh(c2)
        return (h1, c1, h2, c2)

    carry0 = (carry_ref[0], carry_ref[1], carry_ref[2], carry_ref[3])
    h1, c1, h2, c2 = lax.fori_loop(0, _TB, _step, carry0)
    carry_ref[0] = h1
    carry_ref[1] = c1
    carry_ref[2] = h2
    carry_ref[3] = c2

    @pl.when(i == pl.num_programs(0) - 1)
    def _():
        out_ref[...] = jnp.maximum(
            jnp.dot(h2, lw_ref[...], preferred_element_type=jnp.float32)
            + lb_ref[...], 0.0)


def _lstm_tc(seq_p, w1p, u1p, b1p, w2p, b2p, lwp, lbp):
    nb = _N // _TB
    return pl.pallas_call(
        _lstm_tc_body,
        grid=(nb,),
        in_specs=[
            pl.BlockSpec((_TB, 8, 2 * _F), lambda i: (i, 0, 0)),
            pl.BlockSpec((2 * _F, 4 * _G), lambda i: (0, 0)),
            pl.BlockSpec((_F, 4 * _G), lambda i: (0, 0)),
            pl.BlockSpec((1, 4 * _G), lambda i: (0, 0)),
            pl.BlockSpec((2 * _F, 4 * _G), lambda i: (0, 0)),
            pl.BlockSpec((1, 4 * _G), lambda i: (0, 0)),
            pl.BlockSpec((_F, _F), lambda i: (0, 0)),
            pl.BlockSpec((1, _F), lambda i: (0, 0)),
        ],
        out_specs=pl.BlockSpec((8, _F), lambda i: (0, 0)),
        out_shape=jax.ShapeDtypeStruct((8, _F), jnp.float32),
        scratch_shapes=[
            pltpu.VMEM((_TB * 8, 4 * _G), jnp.float32),
            pltpu.VMEM((4, 8, _F), jnp.float32),
        ],
    )(seq_p, w1p, u1p, b1p, w2p, b2p, lwp, lbp)


def _pad_gates(w, h):
    parts = []
    for g in range(4):
        parts.append(jnp.pad(w[:, g * h:(g + 1) * h], ((0, 0), (0, _G - h))))
    return jnp.concatenate(parts, axis=1)


def kernel(X, edge_index, bn1_gamma, bn1_beta, bn2_gamma, bn2_beta,
           lstm1_W, lstm1_U, lstm1_b, lstm2_W, lstm2_U, lstm2_b,
           lin_W, lin_b):
    inv = 1.0 / jnp.sqrt(1.0 + _BN_EPS)

    x_flat = X.reshape(_W * _N, _F)
    src = edge_index[:, 0, :].reshape(-1)
    dst = edge_index[:, 1, :].reshape(-1)
    g1 = (bn1_gamma * inv).reshape(-1)
    g2 = (bn2_gamma * inv).reshape(-1)

    h1_flat, h2_flat = _sc_mpnn(x_flat, src, dst, g1, bn1_beta.reshape(-1),
                                g2, bn2_beta.reshape(-1))

    h1 = h1_flat.reshape(_W, _N, _F).transpose(1, 0, 2)
    h2 = h2_flat.reshape(_W, _N, _F).transpose(1, 0, 2)
    seq_p = jnp.zeros((_N, 8, 2 * _F), jnp.float32)
    seq_p = seq_p.at[:, :_W, :_F].set(h1).at[:, :_W, _F:].set(h2)

    w1p = _pad_gates(lstm1_W, _H)
    u1p = jnp.pad(_pad_gates(lstm1_U, _H), ((0, _F - _H), (0, 0)))
    b1p = _pad_gates(lstm1_b[None, :], _H)
    w2g = _pad_gates(lstm2_W, _H)
    u2g = _pad_gates(lstm2_U, _H)
    w2p = jnp.zeros((2 * _F, 4 * _G), jnp.float32)
    w2p = w2p.at[:_H, :].set(w2g).at[_F:_F + _H, :].set(u2g)
    b2p = _pad_gates(lstm2_b[None, :], _H)
    lwp = jnp.pad(lin_W, ((0, _F - _H), (0, _F - _H)))
    lbp = jnp.pad(lin_b, (0, _F - _H))[None, :]

    out = _lstm_tc(seq_p, w1p, u1p, b1p, w2p, b2p, lwp, lbp)
    return out[:_W, :_H]


# speedup vs baseline: 6.9169x; 1.1744x over previous
import functools

import jax
import jax.numpy as jnp
from jax import lax
from jax.experimental import pallas as pl
from jax.experimental.pallas import tpu as pltpu
from jax.experimental.pallas import tpu_sc as plsc

_BN_EPS = 1e-3

_N = 10000
_E = 160000
_F = 128
_W = 6
_H = 52

_NC = 2
_NS = 16
_TPC = _W // _NC
_EPT = _E // _NS
_K = 80
_NCH = _EPT // _K
_RC = 80
_NCHN = _N // _RC
_CPT = (_NCHN + _NS - 1) // _NS

_TB = 200
_G = 128


def _sc_mpnn_body(x_hbm, src_hbm, dst_hbm, g1_hbm, b1_hbm, g2_hbm, b2_hbm,
                  h1_hbm, h2_hbm,
                  acc, idx_raw, idx_g, rows, zbuf, cbuf, pg, pb, sem):
    c = lax.axis_index("c")
    s = lax.axis_index("s")
    t0 = c * _TPC
    ebase = s * _EPT

    zero16 = jnp.zeros((16,), jnp.float32)

    def _zrow(r, carry):
        for j in range(_F // 16):
            zbuf[r, pl.ds(j * 16, 16)] = zero16
        return carry

    lax.fori_loop(0, _RC, _zrow, 0)

    def _for_my_node_chunks(fn):
        def body(ci, carry):
            cid = s + ci * _NS

            @pl.when(cid < _NCHN)
            def _():
                fn(cid * _RC)

            return carry

        lax.fori_loop(0, _CPT, body, 0)

    def _zero_acc_slice():
        _for_my_node_chunks(
            lambda base: pltpu.sync_copy(zbuf, acc.at[pl.ds(base, _RC)]))

    def _edge_round(table_hbm, toffs, t):
        eoffs = t * _E + ebase

        def _chunk(k, carry):
            off = eoffs + k * _K
            pltpu.sync_copy(src_hbm.at[pl.ds(off, _K)], idx_raw)
            for j in range(_K // 16):
                idx_g[pl.ds(j * 16, 16)] = idx_raw[pl.ds(j * 16, 16)] + toffs
            pltpu.async_copy(table_hbm.at[idx_g], rows, sem).wait()
            pltpu.sync_copy(dst_hbm.at[pl.ds(off, _K)], idx_raw)
            pltpu.sync_copy(rows, acc.at[idx_raw], add=True)
            return carry

        lax.fori_loop(0, _NCH, _chunk, 0)

    def _ew_round(gx_hbm, bx_hbm, out_hbm, toffs, t):
        pltpu.sync_copy(gx_hbm.at[pl.ds(t * _F, _F)], pg)
        pltpu.sync_copy(bx_hbm.at[pl.ds(t * _F, _F)], pb)

        def _one(base):
            pltpu.sync_copy(acc.at[pl.ds(base, _RC)], cbuf)

            def _row(r, carry):
                for j in range(_F // 16):
                    v = cbuf[r, pl.ds(j * 16, 16)]
                    v = jnp.maximum(v, 0.0) * pg[pl.ds(j * 16, 16)] \
                        + pb[pl.ds(j * 16, 16)]
                    cbuf[r, pl.ds(j * 16, 16)] = v
                return carry

            lax.fori_loop(0, _RC, _row, 0)
            pltpu.sync_copy(cbuf, out_hbm.at[pl.ds(toffs + base, _RC)])
            pltpu.sync_copy(zbuf, acc.at[pl.ds(base, _RC)])

        _for_my_node_chunks(_one)

    _zero_acc_slice()
    for j in range(_TPC):
        t = t0 + j
        toffs = t * _N
        plsc.subcore_barrier()
        _edge_round(x_hbm, toffs, t)
        plsc.subcore_barrier()
        _ew_round(g1_hbm, b1_hbm, h1_hbm, toffs, t)
        plsc.subcore_barrier()
        _edge_round(h1_hbm, toffs, t)
        plsc.subcore_barrier()
        _ew_round(g2_hbm, b2_hbm, h2_hbm, toffs, t)


def _sc_mpnn(x_flat, src, dst, g1, b1, g2, b2):
    mesh = plsc.VectorSubcoreMesh(core_axis_name="c", subcore_axis_name="s",
                                  num_cores=_NC, num_subcores=_NS)
    f = pl.kernel(
        _sc_mpnn_body,
        out_type=(
            jax.ShapeDtypeStruct((_W * _N, _F), jnp.float32),
            jax.ShapeDtypeStruct((_W * _N, _F), jnp.float32),
        ),
        mesh=mesh,
        scratch_types=[
            pltpu.VMEM_SHARED((_N, _F), jnp.float32),
            pltpu.VMEM((_K,), jnp.int32),
            pltpu.VMEM((_K,), jnp.int32),
            pltpu.VMEM((_K, _F), jnp.float32),
            pltpu.VMEM((_RC, _F), jnp.float32),
            pltpu.VMEM((_RC, _F), jnp.float32),
            pltpu.VMEM((_F,), jnp.float32),
            pltpu.VMEM((_F,), jnp.float32),
            pltpu.SemaphoreType.DMA,
        ],
    )
    return f(x_flat, src, dst, g1, b1, g2, b2)


def _gates(z, c):
    gi = jax.nn.sigmoid(z[:, 0 * _G:1 * _G])
    gf = jax.nn.sigmoid(z[:, 1 * _G:2 * _G])
    gg = jnp.tanh(z[:, 2 * _G:3 * _G])
    go = jax.nn.sigmoid(z[:, 3 * _G:4 * _G])
    c = gf * c + gi * gg
    h = go * jnp.tanh(c)
    return h, c


def _lstm_tc_body(seq_ref, w1_ref, b1_ref, wa_ref, b2_ref, lw_ref, lb_ref,
                  out_ref, xw_ref, carry_ref):
    i = pl.program_id(0)

    @pl.when(i == 0)
    def _():
        carry_ref[...] = jnp.zeros_like(carry_ref)

    x = seq_ref[...].reshape(_TB * 8, 2 * _F)
    xw_ref[...] = (
        jnp.dot(x, w1_ref[...], preferred_element_type=jnp.float32)
        + b1_ref[...]
    )

    def _step(t, carry):
        h1, c1, h2, c2 = carry
        hcat = jnp.concatenate([h1, h2], axis=1)
        za = jnp.dot(hcat, wa_ref[...], preferred_element_type=jnp.float32)
        xwt = xw_ref[pl.ds(pl.multiple_of(t * 8, 8), 8), :]
        z1 = za[:, :4 * _G] + xwt
        z2 = za[:, 4 * _G:] + b2_ref[...]
        h1, c1 = _gates(z1, c1)
        h2n, c2n = _gates(z2, c2)
        first = (i == 0) & (t == 0)
        h2 = jnp.where(first, 0.0, h2n)
        c2 = jnp.where(first, 0.0, c2n)
        return (h1, c1, h2, c2)

    carry0 = (carry_ref[0], carry_ref[1], carry_ref[2], carry_ref[3])
    h1, c1, h2, c2 = lax.fori_loop(0, _TB, _step, carry0)
    carry_ref[0] = h1
    carry_ref[1] = c1
    carry_ref[2] = h2
    carry_ref[3] = c2

    @pl.when(i == pl.num_programs(0) - 1)
    def _():
        hcat = jnp.concatenate([h1, h2], axis=1)
        za = jnp.dot(hcat, wa_ref[...], preferred_element_type=jnp.float32)
        z2 = za[:, 4 * _G:] + b2_ref[...]
        h2f, _ = _gates(z2, c2)
        out_ref[...] = jnp.maximum(
            jnp.dot(h2f, lw_ref[...], preferred_element_type=jnp.float32)
            + lb_ref[...], 0.0)


def _lstm_tc(seq_p, w1p, b1p, wap, b2p, lwp, lbp):
    nb = _N // _TB
    return pl.pallas_call(
        _lstm_tc_body,
        grid=(nb,),
        in_specs=[
            pl.BlockSpec((_TB, 8, 2 * _F), lambda i: (i, 0, 0)),
            pl.BlockSpec((2 * _F, 4 * _G), lambda i: (0, 0)),
            pl.BlockSpec((1, 4 * _G), lambda i: (0, 0)),
            pl.BlockSpec((2 * _F, 8 * _G), lambda i: (0, 0)),
            pl.BlockSpec((1, 4 * _G), lambda i: (0, 0)),
            pl.BlockSpec((_F, _F), lambda i: (0, 0)),
            pl.BlockSpec((1, _F), lambda i: (0, 0)),
        ],
        out_specs=pl.BlockSpec((8, _F), lambda i: (0, 0)),
        out_shape=jax.ShapeDtypeStruct((8, _F), jnp.float32),
        scratch_shapes=[
            pltpu.VMEM((_TB * 8, 4 * _G), jnp.float32),
            pltpu.VMEM((4, 8, _F), jnp.float32),
        ],
    )(seq_p, w1p, b1p, wap, b2p, lwp, lbp)


def _pad_gates(w, h):
    parts = []
    for g in range(4):
        parts.append(jnp.pad(w[:, g * h:(g + 1) * h], ((0, 0), (0, _G - h))))
    return jnp.concatenate(parts, axis=1)


def kernel(X, edge_index, bn1_gamma, bn1_beta, bn2_gamma, bn2_beta,
           lstm1_W, lstm1_U, lstm1_b, lstm2_W, lstm2_U, lstm2_b,
           lin_W, lin_b):
    inv = 1.0 / jnp.sqrt(1.0 + _BN_EPS)

    x_flat = X.reshape(_W * _N, _F)
    src = edge_index[:, 0, :].reshape(-1)
    dst = edge_index[:, 1, :].reshape(-1)
    g1 = (bn1_gamma * inv).reshape(-1)
    g2 = (bn2_gamma * inv).reshape(-1)

    h1_flat, h2_flat = _sc_mpnn(x_flat, src, dst, g1, bn1_beta.reshape(-1),
                                g2, bn2_beta.reshape(-1))

    h1 = h1_flat.reshape(_W, _N, _F).transpose(1, 0, 2)
    h2 = h2_flat.reshape(_W, _N, _F).transpose(1, 0, 2)
    seq_p = jnp.zeros((_N, 8, 2 * _F), jnp.float32)
    seq_p = seq_p.at[:, :_W, :_F].set(h1).at[:, :_W, _F:].set(h2)

    w1p = _pad_gates(lstm1_W, _H)
    b1p = _pad_gates(lstm1_b[None, :], _H)
    u1g = _pad_gates(lstm1_U, _H)
    w2g = _pad_gates(lstm2_W, _H)
    u2g = _pad_gates(lstm2_U, _H)
    wap = jnp.zeros((2 * _F, 8 * _G), jnp.float32)
    wap = wap.at[:_H, :4 * _G].set(u1g)
    wap = wap.at[:_H, 4 * _G:].set(w2g)
    wap = wap.at[_F:_F + _H, 4 * _G:].set(u2g)
    b2p = _pad_gates(lstm2_b[None, :], _H)
    lwp = jnp.pad(lin_W, ((0, _F - _H), (0, _F - _H)))
    lbp = jnp.pad(lin_b, (0, _F - _H))[None, :]

    out = _lstm_tc(seq_p, w1p, b1p, wap, b2p, lwp, lbp)
    return out[:_W, :_H]
